# Initial kernel scaffold; baseline (speedup 1.0000x reference)
import jax
import jax.numpy as jnp
from jax.experimental import pallas as pl


def kernel(utype_mask, entity_mask, entity_encodings, autoregressive_encoding, self_unit_ct, W_keys, b_keys, W0, b0, W1, b1, Wf, bf, Wi0, bi0, Wi1, bi1, Wo, bo, ln_w, ln_b):
    raise NotImplementedError("write your pallas kernel here")


# speedup vs baseline: 1.9011x; 1.9011x over previous
import jax
import jax.numpy as jnp
from jax.experimental import pallas as pl
from jax.experimental.pallas import tpu as pltpu

N_ENT = 16384
BLK = 2048
NBLK = N_ENT // BLK


def _dot_t(a, b):
    return jax.lax.dot_general(
        a, b, (((1,), (1,)), ((), ())), preferred_element_type=jnp.float32
    )


def _ln(v, w, b):
    mu = jnp.mean(v)
    var = jnp.mean((v - mu) ** 2)
    return (v - mu) / jnp.sqrt(var + 1e-5) * w + b


def _fused_kernel(
    enc_ref, em_ref, ar_ref, wk_ref, bk_ref, w0_ref, b0_ref, w1_ref, b1_ref,
    wf_ref, bf_ref, wi0_ref, bi0_ref, wi1_ref, bi1_ref, wo_ref, bo_ref,
    lnw_ref, lnb_ref, unit_ref, targ_ref, q_sc, stat_sc, idx_sc
):
    i = pl.program_id(0)

    @pl.when(i == 0)
    def _prologue():
        ar = ar_ref[...]
        intermed = _dot_t(ar, w0_ref[...]) + b0_ref[...]
        intermed = jnp.maximum(
            _dot_t(jnp.maximum(intermed, 0.0), w1_ref[...]) + b1_ref[...], 0.0
        )
        x = jnp.concatenate([intermed, jnp.zeros_like(intermed)], axis=1)
        lnw = lnw_ref[...]
        lnb = lnb_ref[...]
        remember = _ln(
            jax.nn.sigmoid(_dot_t(x, wi0_ref[...]) + bi0_ref[...])
            * jnp.tanh(_dot_t(x, wi1_ref[...]) + bi1_ref[...]),
            lnw, lnb,
        )
        out_gate = _ln(jax.nn.sigmoid(_dot_t(x, wo_ref[...]) + bo_ref[...]), lnw, lnb)
        query = jnp.tanh(remember) * out_gate
        q_sc[0:1, 0:32] = query
        stat_sc[0] = 0.0
        stat_sc[1] = -jnp.inf
        idx_sc[0] = 0

    query = q_sc[0:1, 0:32]
    keys = _dot_t(enc_ref[...], wk_ref[...]) + bk_ref[...]
    sim = _dot_t(query, keys)
    logit = jax.nn.sigmoid(sim)
    vec = jnp.exp(jnp.log(logit) / 0.8)
    unit_ref[0:1, pl.ds(i * BLK, BLK)] = vec

    stat_sc[0] += jnp.sum(vec)
    bmax = jnp.max(vec)
    col = jax.lax.broadcasted_iota(jnp.int32, (1, BLK), 1)
    barg = jnp.min(jnp.where(vec == bmax, col, BLK)) + i * BLK

    @pl.when(bmax > stat_sc[1])
    def _update_max():
        stat_sc[1] = bmax
        idx_sc[0] = barg

    @pl.when(i == NBLK - 1)
    def _epilogue():
        s = stat_sc[0]
        pick = idx_sc[0]
        row = unit_ref[...]
        unit_ref[...] = jnp.where(s != 0.0, row / s, row)
        colf = jax.lax.broadcasted_iota(jnp.int32, (1, N_ENT), 1)
        targ_ref[...] = jnp.where(
            (colf == pick) & (em_ref[...] > 0.0), 1.0, 0.0
        )


def kernel(utype_mask, entity_mask, entity_encodings, autoregressive_encoding,
           self_unit_ct, W_keys, b_keys, W0, b0, W1, b1, Wf, bf, Wi0, bi0,
           Wi1, bi1, Wo, bo, ln_w, ln_b):
    em = (1.0 - entity_mask.astype(jnp.float32)).reshape(1, N_ENT)
    ar2 = autoregressive_encoding.reshape(1, 1024)
    row = lambda v: v.reshape(1, -1)

    full = lambda shape: pl.BlockSpec(shape, lambda i: (0, 0))
    unit, targ = pl.pallas_call(
        _fused_kernel,
        grid=(NBLK,),
        in_specs=[
            pl.BlockSpec((BLK, 256), lambda i: (i, 0)),
            full((1, N_ENT)),
            full((1, 1024)),
            full(W_keys.shape),
            full((1, 32)),
            full(W0.shape), full((1, 256)),
            full(W1.shape), full((1, 32)),
            full(Wf.shape), full((1, 32)),
            full(Wi0.shape), full((1, 32)),
            full(Wi1.shape), full((1, 32)),
            full(Wo.shape), full((1, 32)),
            full((1, 32)), full((1, 32)),
        ],
        out_specs=[
            pl.BlockSpec((1, N_ENT), lambda i: (0, 0)),
            pl.BlockSpec((1, N_ENT), lambda i: (0, 0)),
        ],
        out_shape=[
            jax.ShapeDtypeStruct((1, N_ENT), jnp.float32),
            jax.ShapeDtypeStruct((1, N_ENT), jnp.float32),
        ],
        scratch_shapes=[
            pltpu.VMEM((8, 128), jnp.float32),
            pltpu.SMEM((2,), jnp.float32),
            pltpu.SMEM((1,), jnp.int32),
        ],
    )(
        entity_encodings, em, ar2, W_keys, row(b_keys), W0, row(b0),
        W1, row(b1), Wf, row(bf), Wi0, row(bi0), Wi1, row(bi1),
        Wo, row(bo), row(ln_w), row(ln_b)
    )
    return unit, targ.reshape(N_ENT)


# speedup vs baseline: 2.3074x; 1.2137x over previous
import jax
import jax.numpy as jnp
from jax.experimental import pallas as pl
from jax.experimental.pallas import tpu as pltpu

N_ENT = 16384
BLK = 4096
NBLK = N_ENT // BLK


def _dot_t(a, b):
    return jax.lax.dot_general(
        a, b, (((1,), (1,)), ((), ())), preferred_element_type=jnp.float32
    )


def _ln(v, w, b):
    mu = jnp.mean(v)
    var = jnp.mean((v - mu) ** 2)
    return (v - mu) / jnp.sqrt(var + 1e-5) * w + b


def _fused_kernel(
    enc_ref, em_ref, ar_ref, wk_ref, bk_ref, w0_ref, b0_ref, w1_ref, b1_ref,
    wf_ref, bf_ref, wi0_ref, bi0_ref, wi1_ref, bi1_ref, wo_ref, bo_ref,
    lnw_ref, lnb_ref, unit_ref, targ_ref, q_sc, stat_sc, idx_sc
):
    i = pl.program_id(0)

    @pl.when(i == 0)
    def _prologue():
        ar = ar_ref[...]
        intermed = _dot_t(ar, w0_ref[...]) + b0_ref[...]
        intermed = jnp.maximum(
            _dot_t(jnp.maximum(intermed, 0.0), w1_ref[...]) + b1_ref[...], 0.0
        )
        x = jnp.concatenate([intermed, jnp.zeros_like(intermed)], axis=1)
        lnw = lnw_ref[...]
        lnb = lnb_ref[...]
        remember = _ln(
            jax.nn.sigmoid(_dot_t(x, wi0_ref[...]) + bi0_ref[...])
            * jnp.tanh(_dot_t(x, wi1_ref[...]) + bi1_ref[...]),
            lnw, lnb,
        )
        out_gate = _ln(jax.nn.sigmoid(_dot_t(x, wo_ref[...]) + bo_ref[...]), lnw, lnb)
        query = jnp.tanh(remember) * out_gate
        q_sc[0:1, 0:32] = query
        stat_sc[0] = 0.0
        stat_sc[1] = -jnp.inf
        idx_sc[0] = 0

    query = q_sc[0:1, 0:32]
    keys = _dot_t(enc_ref[...], wk_ref[...]) + bk_ref[...]
    sim = _dot_t(query, keys)
    logit = jax.nn.sigmoid(sim)
    vec = jnp.exp(jnp.log(logit) / 0.8)
    unit_ref[0:1, pl.ds(i * BLK, BLK)] = vec

    stat_sc[0] += jnp.sum(vec)
    bmax = jnp.max(vec)
    col = jax.lax.broadcasted_iota(jnp.int32, (1, BLK), 1)
    barg = jnp.min(jnp.where(vec == bmax, col, BLK)) + i * BLK

    @pl.when(bmax > stat_sc[1])
    def _update_max():
        stat_sc[1] = bmax
        idx_sc[0] = barg

    @pl.when(i == NBLK - 1)
    def _epilogue():
        s = stat_sc[0]
        pick = idx_sc[0]
        row = unit_ref[...]
        unit_ref[...] = jnp.where(s != 0.0, row / s, row)
        colf = jax.lax.broadcasted_iota(jnp.int32, (1, N_ENT), 1)
        targ_ref[...] = jnp.where(
            (colf == pick) & (em_ref[...] > 0.0), 1.0, 0.0
        )


def kernel(utype_mask, entity_mask, entity_encodings, autoregressive_encoding,
           self_unit_ct, W_keys, b_keys, W0, b0, W1, b1, Wf, bf, Wi0, bi0,
           Wi1, bi1, Wo, bo, ln_w, ln_b):
    em = (1.0 - entity_mask.astype(jnp.float32)).reshape(1, N_ENT)
    ar2 = autoregressive_encoding.reshape(1, 1024)
    row = lambda v: v.reshape(1, -1)

    full = lambda shape: pl.BlockSpec(shape, lambda i: (0, 0))
    unit, targ = pl.pallas_call(
        _fused_kernel,
        grid=(NBLK,),
        in_specs=[
            pl.BlockSpec((BLK, 256), lambda i: (i, 0)),
            full((1, N_ENT)),
            full((1, 1024)),
            full(W_keys.shape),
            full((1, 32)),
            full(W0.shape), full((1, 256)),
            full(W1.shape), full((1, 32)),
            full(Wf.shape), full((1, 32)),
            full(Wi0.shape), full((1, 32)),
            full(Wi1.shape), full((1, 32)),
            full(Wo.shape), full((1, 32)),
            full((1, 32)), full((1, 32)),
        ],
        out_specs=[
            pl.BlockSpec((1, N_ENT), lambda i: (0, 0)),
            pl.BlockSpec((1, N_ENT), lambda i: (0, 0)),
        ],
        out_shape=[
            jax.ShapeDtypeStruct((1, N_ENT), jnp.float32),
            jax.ShapeDtypeStruct((1, N_ENT), jnp.float32),
        ],
        scratch_shapes=[
            pltpu.VMEM((8, 128), jnp.float32),
            pltpu.SMEM((2,), jnp.float32),
            pltpu.SMEM((1,), jnp.int32),
        ],
    )(
        entity_encodings, em, ar2, W_keys, row(b_keys), W0, row(b0),
        W1, row(b1), Wf, row(bf), Wi0, row(bi0), Wi1, row(bi1),
        Wo, row(bo), row(ln_w), row(ln_b)
    )
    return unit, targ.reshape(N_ENT)


# speedup vs baseline: 2.4039x; 1.0418x over previous
import jax
import jax.numpy as jnp
from jax.experimental import pallas as pl
from jax.experimental.pallas import tpu as pltpu

N_ENT = 16384
BLK = 8192
NBLK = N_ENT // BLK


def _dot_t(a, b):
    return jax.lax.dot_general(
        a, b, (((1,), (1,)), ((), ())), preferred_element_type=jnp.float32
    )


def _ln(v, w, b):
    mu = jnp.mean(v)
    var = jnp.mean((v - mu) ** 2)
    return (v - mu) / jnp.sqrt(var + 1e-5) * w + b


def _fused_kernel(
    enc_ref, em_ref, ar_ref, wk_ref, bk_ref, w0_ref, b0_ref, w1_ref, b1_ref,
    wf_ref, bf_ref, wi0_ref, bi0_ref, wi1_ref, bi1_ref, wo_ref, bo_ref,
    lnw_ref, lnb_ref, unit_ref, targ_ref, q_sc, stat_sc, idx_sc
):
    i = pl.program_id(0)

    @pl.when(i == 0)
    def _prologue():
        ar = ar_ref[...]
        intermed = _dot_t(ar, w0_ref[...]) + b0_ref[...]
        intermed = jnp.maximum(
            _dot_t(jnp.maximum(intermed, 0.0), w1_ref[...]) + b1_ref[...], 0.0
        )
        x = jnp.concatenate([intermed, jnp.zeros_like(intermed)], axis=1)
        lnw = lnw_ref[...]
        lnb = lnb_ref[...]
        remember = _ln(
            jax.nn.sigmoid(_dot_t(x, wi0_ref[...]) + bi0_ref[...])
            * jnp.tanh(_dot_t(x, wi1_ref[...]) + bi1_ref[...]),
            lnw, lnb,
        )
        out_gate = _ln(jax.nn.sigmoid(_dot_t(x, wo_ref[...]) + bo_ref[...]), lnw, lnb)
        query = jnp.tanh(remember) * out_gate
        q_sc[0:1, 0:32] = query
        stat_sc[0] = 0.0
        stat_sc[1] = -jnp.inf
        idx_sc[0] = 0

    query = q_sc[0:1, 0:32]
    keys = _dot_t(enc_ref[...], wk_ref[...]) + bk_ref[...]
    sim = _dot_t(query, keys)
    logit = jax.nn.sigmoid(sim)
    vec = jnp.exp(jnp.log(logit) / 0.8)
    unit_ref[0:1, pl.ds(i * BLK, BLK)] = vec

    stat_sc[0] += jnp.sum(vec)
    bmax = jnp.max(vec)
    col = jax.lax.broadcasted_iota(jnp.int32, (1, BLK), 1)
    barg = jnp.min(jnp.where(vec == bmax, col, BLK)) + i * BLK

    @pl.when(bmax > stat_sc[1])
    def _update_max():
        stat_sc[1] = bmax
        idx_sc[0] = barg

    @pl.when(i == NBLK - 1)
    def _epilogue():
        s = stat_sc[0]
        pick = idx_sc[0]
        row = unit_ref[...]
        unit_ref[...] = jnp.where(s != 0.0, row / s, row)
        colf = jax.lax.broadcasted_iota(jnp.int32, (1, N_ENT), 1)
        targ_ref[...] = jnp.where(
            (colf == pick) & (em_ref[...] > 0.0), 1.0, 0.0
        )


def kernel(utype_mask, entity_mask, entity_encodings, autoregressive_encoding,
           self_unit_ct, W_keys, b_keys, W0, b0, W1, b1, Wf, bf, Wi0, bi0,
           Wi1, bi1, Wo, bo, ln_w, ln_b):
    em = (1.0 - entity_mask.astype(jnp.float32)).reshape(1, N_ENT)
    ar2 = autoregressive_encoding.reshape(1, 1024)
    row = lambda v: v.reshape(1, -1)

    full = lambda shape: pl.BlockSpec(shape, lambda i: (0, 0))
    unit, targ = pl.pallas_call(
        _fused_kernel,
        grid=(NBLK,),
        in_specs=[
            pl.BlockSpec((BLK, 256), lambda i: (i, 0)),
            full((1, N_ENT)),
            full((1, 1024)),
            full(W_keys.shape),
            full((1, 32)),
            full(W0.shape), full((1, 256)),
            full(W1.shape), full((1, 32)),
            full(Wf.shape), full((1, 32)),
            full(Wi0.shape), full((1, 32)),
            full(Wi1.shape), full((1, 32)),
            full(Wo.shape), full((1, 32)),
            full((1, 32)), full((1, 32)),
        ],
        out_specs=[
            pl.BlockSpec((1, N_ENT), lambda i: (0, 0)),
            pl.BlockSpec((1, N_ENT), lambda i: (0, 0)),
        ],
        out_shape=[
            jax.ShapeDtypeStruct((1, N_ENT), jnp.float32),
            jax.ShapeDtypeStruct((1, N_ENT), jnp.float32),
        ],
        scratch_shapes=[
            pltpu.VMEM((8, 128), jnp.float32),
            pltpu.SMEM((2,), jnp.float32),
            pltpu.SMEM((1,), jnp.int32),
        ],
    )(
        entity_encodings, em, ar2, W_keys, row(b_keys), W0, row(b0),
        W1, row(b1), Wf, row(bf), Wi0, row(bi0), Wi1, row(bi1),
        Wo, row(bo), row(ln_w), row(ln_b)
    )
    return unit, targ.reshape(N_ENT)
